# baseline (speedup 1.0000x reference)
import functools

import jax
import jax.numpy as jnp
from jax import lax
from jax.experimental import pallas as pl
from jax.experimental.pallas import tpu as pltpu
from jax.experimental.pallas import tpu_sc as plsc

E = 64
TOPK = 2
DIM = 1024
FFN = 512
SEQ = 2048
N = SEQ * TOPK
T = 128
PADN = N + E * T
NT = PADN // T

NC = 2
NS = 16
NW = NC * NS
TPW = SEQ // NW
CT = 32
LW = 16


def _route_body(x_ref, gw_ref, xbf_ref, pos0_ref, pos1_ref, w_ref, eid_ref,
                nact_ref):
    x = x_ref[...]
    gw = gw_ref[...]
    xbf_ref[...] = x.astype(jnp.bfloat16)
    logits = lax.dot_general(x, gw, (((1,), (1,)), ((), ())),
                             preferred_element_type=jnp.float32)
    m = jnp.max(logits, axis=1, keepdims=True)
    ex = jnp.exp(logits - m)
    scores = ex / jnp.sum(ex, axis=1, keepdims=True)

    eidx = lax.broadcasted_iota(jnp.int32, (SEQ, E), 1)
    m1 = jnp.max(scores, axis=1, keepdims=True)
    a1 = jnp.min(jnp.where(scores == m1, eidx, E), axis=1, keepdims=True)
    s2 = jnp.where(eidx == a1, -jnp.inf, scores)
    m2 = jnp.max(s2, axis=1, keepdims=True)
    a2 = jnp.min(jnp.where(s2 == m2, eidx, E), axis=1, keepdims=True)

    denom = m1 + m2 + 1e-20
    w_ref[:, :LW] = jnp.broadcast_to(m1 / denom, (SEQ, LW))
    w_ref[:, LW:] = jnp.broadcast_to(m2 / denom, (SEQ, LW))

    one0 = (eidx == a1).astype(jnp.float32)
    one1 = (eidx == a2).astype(jnp.float32)
    both = one0 + one1

    counts = jnp.sum(both, axis=0, keepdims=True)
    pc = ((counts.astype(jnp.int32) + (T - 1)) // T) * T
    pcf = pc.astype(jnp.float32)
    er = lax.broadcasted_iota(jnp.int32, (E, E), 0)
    ec = lax.broadcasted_iota(jnp.int32, (E, E), 1)
    upper = (er < ec).astype(jnp.float32)
    poff = lax.dot_general(pcf, upper, (((1,), (0,)), ((), ())),
                           preferred_element_type=jnp.float32)
    total = jnp.sum(pcf, axis=1, keepdims=True)

    B = 512
    br = lax.broadcasted_iota(jnp.int32, (B, B), 0)
    bc = lax.broadcasted_iota(jnp.int32, (B, B), 1)
    ltri = (bc < br).astype(jnp.float32)
    carry = jnp.zeros((1, E), jnp.float32)
    for b in range(SEQ // B):
        sl = slice(b * B, (b + 1) * B)
        cb = both[sl]
        run = lax.dot_general(ltri, cb, (((1,), (0,)), ((), ())),
                              preferred_element_type=jnp.float32) + carry
        carry = carry + jnp.sum(cb, axis=0, keepdims=True)
        dest = run + poff
        pos0_ref[sl, :] = jnp.sum(one0[sl] * dest, axis=1,
                                  keepdims=True).astype(jnp.int32)
        pos1_ref[sl, :] = jnp.sum(one1[sl] * dest, axis=1,
                                  keepdims=True).astype(jnp.int32)

    tstart = (lax.broadcasted_iota(jnp.int32, (NT, 1), 0) * T).astype(jnp.float32)
    p = jnp.minimum(tstart, total - 1.0)
    eid_ref[...] = jnp.sum((poff <= p).astype(jnp.int32), axis=1,
                           keepdims=True) - 1
    nact_ref[...] = (total.astype(jnp.int32) // T)


_route = pl.pallas_call(
    _route_body,
    out_shape=(
        jax.ShapeDtypeStruct((SEQ, DIM), jnp.bfloat16),
        jax.ShapeDtypeStruct((SEQ, 1), jnp.int32),
        jax.ShapeDtypeStruct((SEQ, 1), jnp.int32),
        jax.ShapeDtypeStruct((SEQ, TOPK * LW), jnp.float32),
        jax.ShapeDtypeStruct((NT, 1), jnp.int32),
        jax.ShapeDtypeStruct((1, 1), jnp.int32),
    ),
)


@functools.lru_cache(maxsize=None)
def _sc_kernels():
    mesh = plsc.VectorSubcoreMesh(core_axis_name="c", subcore_axis_name="s")

    @functools.partial(
        pl.kernel,
        out_type=jax.ShapeDtypeStruct((PADN, DIM // 2), jnp.int32),
        mesh=mesh,
        scratch_types=[
            pltpu.VMEM((TPW,), jnp.int32),
            pltpu.VMEM((TPW,), jnp.int32),
            pltpu.VMEM((TPW, DIM // 2), jnp.int32),
            pltpu.SemaphoreType.DMA,
        ],
    )
    def _dispatch(x_hbm, pos0_hbm, pos1_hbm, out_hbm, d0_v, d1_v, rows_v, sem):
        wid = lax.axis_index("s") * NC + lax.axis_index("c")
        base = wid * TPW
        pltpu.sync_copy(pos0_hbm.at[pl.ds(base, TPW)], d0_v)
        pltpu.sync_copy(pos1_hbm.at[pl.ds(base, TPW)], d1_v)
        pltpu.sync_copy(x_hbm.at[pl.ds(base, TPW)], rows_v)
        c0 = pltpu.async_copy(rows_v, out_hbm.at[d0_v], sem)
        c1 = pltpu.async_copy(rows_v, out_hbm.at[d1_v], sem)
        c0.wait()
        c1.wait()

    @functools.partial(
        pl.kernel,
        out_type=jax.ShapeDtypeStruct((SEQ, DIM), jnp.float32),
        mesh=mesh,
        scratch_types=[
            pltpu.VMEM((CT,), jnp.int32),
            pltpu.VMEM((CT,), jnp.int32),
            pltpu.VMEM((CT, LW), jnp.float32),
            pltpu.VMEM((CT, LW), jnp.float32),
            pltpu.VMEM((CT, DIM), jnp.float32),
            pltpu.VMEM((CT, DIM), jnp.float32),
            pltpu.VMEM((CT, DIM), jnp.float32),
            pltpu.SemaphoreType.DMA,
            pltpu.SemaphoreType.DMA,
        ],
    )
    def _comb(ffn_hbm, pos0_hbm, pos1_hbm, w0_hbm, w1_hbm, out_hbm,
              i0_v, i1_v, w0_v, w1_v, ra_v, rb_v, ro_v, sema, semb):
        wid = lax.axis_index("s") * NC + lax.axis_index("c")
        for c in range(TPW // CT):
            base = wid * TPW + c * CT
            pltpu.sync_copy(pos0_hbm.at[pl.ds(base, CT)], i0_v)
            pltpu.sync_copy(pos1_hbm.at[pl.ds(base, CT)], i1_v)
            pltpu.sync_copy(w0_hbm.at[pl.ds(base, CT)], w0_v)
            pltpu.sync_copy(w1_hbm.at[pl.ds(base, CT)], w1_v)
            ca = pltpu.async_copy(ffn_hbm.at[i0_v], ra_v, sema)
            cb = pltpu.async_copy(ffn_hbm.at[i1_v], rb_v, semb)
            ca.wait()
            cb.wait()

            @plsc.parallel_loop(0, CT)
            def _tok(j):
                wa = w0_v[j]
                wb = w1_v[j]
                for k in range(DIM // LW):
                    sl = pl.ds(k * LW, LW)
                    ro_v[j, sl] = wa * ra_v[j, sl] + wb * rb_v[j, sl]

            pltpu.sync_copy(ro_v, out_hbm.at[pl.ds(base, CT)])

    return _dispatch, _comb


def _gemm_body(eid_ref, nact_ref, x_ref, wg_ref, wu_ref, wd_ref, o_ref):
    t = pl.program_id(0)

    @pl.when(t < nact_ref[0])
    def _():
        xt = x_ref[...].astype(jnp.float32)
        g = lax.dot_general(xt, wg_ref[0], (((1,), (1,)), ((), ())),
                            preferred_element_type=jnp.float32)
        u = lax.dot_general(xt, wu_ref[0], (((1,), (1,)), ((), ())),
                            preferred_element_type=jnp.float32)
        h = g * (1.0 / (1.0 + jnp.exp(-g))) * u
        o_ref[...] = lax.dot_general(h, wd_ref[0], (((1,), (1,)), ((), ())),
                                     preferred_element_type=jnp.float32)


_gemm = pl.pallas_call(
    _gemm_body,
    grid_spec=pltpu.PrefetchScalarGridSpec(
        num_scalar_prefetch=2,
        grid=(NT,),
        in_specs=[
            pl.BlockSpec((T, DIM),
                         lambda t, eid, na: (jnp.minimum(t, na[0] - 1), 0)),
            pl.BlockSpec((1, FFN, DIM), lambda t, eid, na: (eid[t], 0, 0)),
            pl.BlockSpec((1, FFN, DIM), lambda t, eid, na: (eid[t], 0, 0)),
            pl.BlockSpec((1, DIM, FFN), lambda t, eid, na: (eid[t], 0, 0)),
        ],
        out_specs=pl.BlockSpec(
            (T, DIM), lambda t, eid, na: (jnp.where(t < na[0], t, NT - 1), 0)),
    ),
    out_shape=jax.ShapeDtypeStruct((PADN, DIM), jnp.float32),
    compiler_params=pltpu.CompilerParams(
        dimension_semantics=("arbitrary",)),
)


def kernel(hidden_states, gate_weight, gate_proj_w, up_proj_w, down_proj_w):
    b, s, h = hidden_states.shape
    x = hidden_states.reshape(SEQ, DIM).astype(jnp.float32)
    xbf, pos0, pos1, wexp, eid2, nact2 = _route(x, gate_weight)
    pos0 = pos0.reshape(SEQ)
    pos1 = pos1.reshape(SEQ)
    _dispatch, _comb = _sc_kernels()
    xi = lax.bitcast_convert_type(xbf.reshape(SEQ, DIM // 2, 2), jnp.int32)
    sorted_i = _dispatch(xi, pos0, pos1)
    sorted_x = lax.bitcast_convert_type(sorted_i, jnp.bfloat16)
    sorted_x = sorted_x.reshape(PADN, DIM)
    ffn = _gemm(eid2.reshape(NT), nact2.reshape(1), sorted_x,
                gate_proj_w, up_proj_w, down_proj_w)
    out = _comb(ffn, pos0, pos1, wexp[:, :LW], wexp[:, LW:])
    return out.reshape(b, s, h)


# speedup vs baseline: 2.4210x; 2.4210x over previous
import functools

import jax
import jax.numpy as jnp
from jax import lax
from jax.experimental import pallas as pl
from jax.experimental.pallas import tpu as pltpu
from jax.experimental.pallas import tpu_sc as plsc

E = 64
TOPK = 2
DIM = 1024
FFN = 512
SEQ = 2048
N = SEQ * TOPK
T = 128
PADN = N + E * T
NT = PADN // T

NC = 2
NS = 16
NW = NC * NS
TPW = SEQ // NW
CT = 32
LW = 16


def _route_body(x_ref, gw_ref, xp_ref, pos0_ref, pos1_ref, w_ref, eid_ref,
                nact_ref):
    x = x_ref[...]
    gw = gw_ref[...]
    lo = lax.bitcast_convert_type(
        x[:, :DIM // 2].astype(jnp.bfloat16), jnp.int16).astype(jnp.int32)
    hi = lax.bitcast_convert_type(
        x[:, DIM // 2:].astype(jnp.bfloat16), jnp.int16).astype(jnp.int32)
    xp_ref[...] = (lo & 0xFFFF) | (hi << 16)
    logits = lax.dot_general(x, gw, (((1,), (1,)), ((), ())),
                             preferred_element_type=jnp.float32)
    m = jnp.max(logits, axis=1, keepdims=True)
    ex = jnp.exp(logits - m)
    scores = ex / jnp.sum(ex, axis=1, keepdims=True)

    eidx = lax.broadcasted_iota(jnp.int32, (SEQ, E), 1)
    m1 = jnp.max(scores, axis=1, keepdims=True)
    a1 = jnp.min(jnp.where(scores == m1, eidx, E), axis=1, keepdims=True)
    s2 = jnp.where(eidx == a1, -jnp.inf, scores)
    m2 = jnp.max(s2, axis=1, keepdims=True)
    a2 = jnp.min(jnp.where(s2 == m2, eidx, E), axis=1, keepdims=True)

    denom = m1 + m2 + 1e-20
    w_ref[:, :LW] = jnp.broadcast_to(m1 / denom, (SEQ, LW))
    w_ref[:, LW:] = jnp.broadcast_to(m2 / denom, (SEQ, LW))

    one0 = (eidx == a1).astype(jnp.float32)
    one1 = (eidx == a2).astype(jnp.float32)
    both = one0 + one1

    counts = jnp.sum(both, axis=0, keepdims=True)
    pc = ((counts.astype(jnp.int32) + (T - 1)) // T) * T
    pcf = pc.astype(jnp.float32)
    er = lax.broadcasted_iota(jnp.int32, (E, E), 0)
    ec = lax.broadcasted_iota(jnp.int32, (E, E), 1)
    upper = (er < ec).astype(jnp.float32)
    poff = lax.dot_general(pcf, upper, (((1,), (0,)), ((), ())),
                           preferred_element_type=jnp.float32)
    total = jnp.sum(pcf, axis=1, keepdims=True)

    B = 512
    br = lax.broadcasted_iota(jnp.int32, (B, B), 0)
    bc = lax.broadcasted_iota(jnp.int32, (B, B), 1)
    ltri = (bc < br).astype(jnp.float32)
    carry = jnp.zeros((1, E), jnp.float32)
    for b in range(SEQ // B):
        sl = slice(b * B, (b + 1) * B)
        cb = both[sl]
        run = lax.dot_general(ltri, cb, (((1,), (0,)), ((), ())),
                              preferred_element_type=jnp.float32) + carry
        carry = carry + jnp.sum(cb, axis=0, keepdims=True)
        dest = run + poff
        pos0_ref[sl, :] = jnp.sum(one0[sl] * dest, axis=1,
                                  keepdims=True).astype(jnp.int32)
        pos1_ref[sl, :] = jnp.sum(one1[sl] * dest, axis=1,
                                  keepdims=True).astype(jnp.int32)

    tstart = (lax.broadcasted_iota(jnp.int32, (NT, 1), 0) * T).astype(jnp.float32)
    p = jnp.minimum(tstart, total - 1.0)
    eid_ref[...] = jnp.sum((poff <= p).astype(jnp.int32), axis=1,
                           keepdims=True) - 1
    nact_ref[...] = (total.astype(jnp.int32) // T)


_route = pl.pallas_call(
    _route_body,
    out_shape=(
        jax.ShapeDtypeStruct((SEQ, DIM // 2), jnp.int32),
        jax.ShapeDtypeStruct((SEQ, 1), jnp.int32),
        jax.ShapeDtypeStruct((SEQ, 1), jnp.int32),
        jax.ShapeDtypeStruct((SEQ, TOPK * LW), jnp.float32),
        jax.ShapeDtypeStruct((NT, 1), jnp.int32),
        jax.ShapeDtypeStruct((1, 1), jnp.int32),
    ),
)


@functools.lru_cache(maxsize=None)
def _sc_kernels():
    mesh = plsc.VectorSubcoreMesh(core_axis_name="c", subcore_axis_name="s")

    @functools.partial(
        pl.kernel,
        out_type=jax.ShapeDtypeStruct((PADN, DIM // 2), jnp.int32),
        mesh=mesh,
        scratch_types=[
            pltpu.VMEM((TPW,), jnp.int32),
            pltpu.VMEM((TPW,), jnp.int32),
            pltpu.VMEM((TPW, DIM // 2), jnp.int32),
            pltpu.SemaphoreType.DMA,
        ],
    )
    def _dispatch(x_hbm, pos0_hbm, pos1_hbm, out_hbm, d0_v, d1_v, rows_v, sem):
        wid = lax.axis_index("s") * NC + lax.axis_index("c")
        base = wid * TPW
        pltpu.sync_copy(pos0_hbm.at[pl.ds(base, TPW)], d0_v)
        pltpu.sync_copy(pos1_hbm.at[pl.ds(base, TPW)], d1_v)
        pltpu.sync_copy(x_hbm.at[pl.ds(base, TPW)], rows_v)
        c0 = pltpu.async_copy(rows_v, out_hbm.at[d0_v], sem)
        c1 = pltpu.async_copy(rows_v, out_hbm.at[d1_v], sem)
        c0.wait()
        c1.wait()

    @functools.partial(
        pl.kernel,
        out_type=jax.ShapeDtypeStruct((SEQ, DIM), jnp.float32),
        mesh=mesh,
        scratch_types=[
            pltpu.VMEM((CT,), jnp.int32),
            pltpu.VMEM((CT,), jnp.int32),
            pltpu.VMEM((CT, LW), jnp.float32),
            pltpu.VMEM((CT, LW), jnp.float32),
            pltpu.VMEM((CT, DIM), jnp.float32),
            pltpu.VMEM((CT, DIM), jnp.float32),
            pltpu.VMEM((CT, DIM), jnp.float32),
            pltpu.SemaphoreType.DMA,
            pltpu.SemaphoreType.DMA,
        ],
    )
    def _comb(ffn_hbm, pos0_hbm, pos1_hbm, w0_hbm, w1_hbm, out_hbm,
              i0_v, i1_v, w0_v, w1_v, ra_v, rb_v, ro_v, sema, semb):
        wid = lax.axis_index("s") * NC + lax.axis_index("c")
        for c in range(TPW // CT):
            base = wid * TPW + c * CT
            pltpu.sync_copy(pos0_hbm.at[pl.ds(base, CT)], i0_v)
            pltpu.sync_copy(pos1_hbm.at[pl.ds(base, CT)], i1_v)
            pltpu.sync_copy(w0_hbm.at[pl.ds(base, CT)], w0_v)
            pltpu.sync_copy(w1_hbm.at[pl.ds(base, CT)], w1_v)
            ca = pltpu.async_copy(ffn_hbm.at[i0_v], ra_v, sema)
            cb = pltpu.async_copy(ffn_hbm.at[i1_v], rb_v, semb)
            ca.wait()
            cb.wait()

            @plsc.parallel_loop(0, CT)
            def _tok(j):
                wa = w0_v[j]
                wb = w1_v[j]
                for k in range(DIM // LW):
                    sl = pl.ds(k * LW, LW)
                    ro_v[j, sl] = wa * ra_v[j, sl] + wb * rb_v[j, sl]

            pltpu.sync_copy(ro_v, out_hbm.at[pl.ds(base, CT)])

    return _dispatch, _comb


def _gemm_body(eid_ref, nact_ref, x_ref, wg_ref, wu_ref, wd_ref, o_ref):
    t = pl.program_id(0)

    @pl.when(t < nact_ref[0])
    def _():
        xw = x_ref[...]
        lo = lax.bitcast_convert_type(
            xw.astype(jnp.int16), jnp.bfloat16).astype(jnp.float32)
        hi = lax.bitcast_convert_type(
            (xw >> 16).astype(jnp.int16), jnp.bfloat16).astype(jnp.float32)
        xt = jnp.concatenate([lo, hi], axis=1)
        g = lax.dot_general(xt, wg_ref[0], (((1,), (1,)), ((), ())),
                            preferred_element_type=jnp.float32)
        u = lax.dot_general(xt, wu_ref[0], (((1,), (1,)), ((), ())),
                            preferred_element_type=jnp.float32)
        h = g * (1.0 / (1.0 + jnp.exp(-g))) * u
        o_ref[...] = lax.dot_general(h, wd_ref[0], (((1,), (1,)), ((), ())),
                                     preferred_element_type=jnp.float32)


_gemm = pl.pallas_call(
    _gemm_body,
    grid_spec=pltpu.PrefetchScalarGridSpec(
        num_scalar_prefetch=2,
        grid=(NT,),
        in_specs=[
            pl.BlockSpec((T, DIM // 2),
                         lambda t, eid, na: (jnp.minimum(t, na[0] - 1), 0)),
            pl.BlockSpec((1, FFN, DIM), lambda t, eid, na: (eid[t], 0, 0)),
            pl.BlockSpec((1, FFN, DIM), lambda t, eid, na: (eid[t], 0, 0)),
            pl.BlockSpec((1, DIM, FFN), lambda t, eid, na: (eid[t], 0, 0)),
        ],
        out_specs=pl.BlockSpec(
            (T, DIM), lambda t, eid, na: (jnp.where(t < na[0], t, NT - 1), 0)),
    ),
    out_shape=jax.ShapeDtypeStruct((PADN, DIM), jnp.float32),
    compiler_params=pltpu.CompilerParams(
        dimension_semantics=("arbitrary",)),
)


def kernel(hidden_states, gate_weight, gate_proj_w, up_proj_w, down_proj_w):
    b, s, h = hidden_states.shape
    x = hidden_states.reshape(SEQ, DIM).astype(jnp.float32)
    xp, pos0, pos1, wexp, eid2, nact2 = _route(x, gate_weight)
    pos0 = pos0.reshape(SEQ)
    pos1 = pos1.reshape(SEQ)
    _dispatch, _comb = _sc_kernels()
    sorted_x = _dispatch(xp, pos0, pos1)
    ffn = _gemm(eid2.reshape(NT), nact2.reshape(1), sorted_x,
                gate_proj_w, up_proj_w, down_proj_w)
    out = _comb(ffn, pos0, pos1, wexp[:, :LW], wexp[:, LW:])
    return out.reshape(b, s, h)


# speedup vs baseline: 2.4829x; 1.0255x over previous
import functools

import jax
import jax.numpy as jnp
from jax import lax
from jax.experimental import pallas as pl
from jax.experimental.pallas import tpu as pltpu
from jax.experimental.pallas import tpu_sc as plsc

E = 64
TOPK = 2
DIM = 1024
FFN = 512
SEQ = 2048
N = SEQ * TOPK
T = 128
PADN = N + E * T
NT = PADN // T

NC = 2
NS = 16
NW = NC * NS
TPW = SEQ // NW
CT = 32
LW = 16


def _route_body(x_ref, gw_ref, xp_ref, pos0_ref, pos1_ref, w_ref, eid_ref,
                nact_ref):
    x = x_ref[...]
    gw = gw_ref[...]
    lo = lax.bitcast_convert_type(
        x[:, :DIM // 2].astype(jnp.bfloat16), jnp.int16).astype(jnp.int32)
    hi = lax.bitcast_convert_type(
        x[:, DIM // 2:].astype(jnp.bfloat16), jnp.int16).astype(jnp.int32)
    xp_ref[...] = (lo & 0xFFFF) | (hi << 16)
    logits = lax.dot_general(x, gw, (((1,), (1,)), ((), ())),
                             preferred_element_type=jnp.float32)
    m = jnp.max(logits, axis=1, keepdims=True)
    ex = jnp.exp(logits - m)
    scores = ex / jnp.sum(ex, axis=1, keepdims=True)

    eidx = lax.broadcasted_iota(jnp.int32, (SEQ, E), 1)
    m1 = jnp.max(scores, axis=1, keepdims=True)
    a1 = jnp.min(jnp.where(scores == m1, eidx, E), axis=1, keepdims=True)
    s2 = jnp.where(eidx == a1, -jnp.inf, scores)
    m2 = jnp.max(s2, axis=1, keepdims=True)
    a2 = jnp.min(jnp.where(s2 == m2, eidx, E), axis=1, keepdims=True)

    denom = m1 + m2 + 1e-20
    w_ref[:, :LW] = jnp.broadcast_to(m1 / denom, (SEQ, LW))
    w_ref[:, LW:] = jnp.broadcast_to(m2 / denom, (SEQ, LW))

    one0 = (eidx == a1).astype(jnp.float32)
    one1 = (eidx == a2).astype(jnp.float32)
    both = one0 + one1

    counts = jnp.sum(both, axis=0, keepdims=True)
    pc = ((counts.astype(jnp.int32) + (T - 1)) // T) * T
    pcf = pc.astype(jnp.float32)
    er = lax.broadcasted_iota(jnp.int32, (E, E), 0)
    ec = lax.broadcasted_iota(jnp.int32, (E, E), 1)
    upper = (er < ec).astype(jnp.float32)
    poff = lax.dot_general(pcf, upper, (((1,), (0,)), ((), ())),
                           preferred_element_type=jnp.float32)
    total = jnp.sum(pcf, axis=1, keepdims=True)

    B = 512
    br = lax.broadcasted_iota(jnp.int32, (B, B), 0)
    bc = lax.broadcasted_iota(jnp.int32, (B, B), 1)
    ltri = (bc < br).astype(jnp.float32)
    carry = jnp.zeros((1, E), jnp.float32)
    for b in range(SEQ // B):
        sl = slice(b * B, (b + 1) * B)
        cb = both[sl]
        run = lax.dot_general(ltri, cb, (((1,), (0,)), ((), ())),
                              preferred_element_type=jnp.float32) + carry
        carry = carry + jnp.sum(cb, axis=0, keepdims=True)
        dest = run + poff
        pos0_ref[sl, :] = jnp.sum(one0[sl] * dest, axis=1,
                                  keepdims=True).astype(jnp.int32)
        pos1_ref[sl, :] = jnp.sum(one1[sl] * dest, axis=1,
                                  keepdims=True).astype(jnp.int32)

    tstart = (lax.broadcasted_iota(jnp.int32, (NT, 1), 0) * T).astype(jnp.float32)
    p = jnp.minimum(tstart, total - 1.0)
    eid_ref[...] = jnp.sum((poff <= p).astype(jnp.int32), axis=1,
                           keepdims=True) - 1
    nact_ref[...] = (total.astype(jnp.int32) // T)


_route = pl.pallas_call(
    _route_body,
    out_shape=(
        jax.ShapeDtypeStruct((SEQ, DIM // 2), jnp.int32),
        jax.ShapeDtypeStruct((SEQ, 1), jnp.int32),
        jax.ShapeDtypeStruct((SEQ, 1), jnp.int32),
        jax.ShapeDtypeStruct((SEQ, TOPK * LW), jnp.float32),
        jax.ShapeDtypeStruct((NT, 1), jnp.int32),
        jax.ShapeDtypeStruct((1, 1), jnp.int32),
    ),
)


@functools.lru_cache(maxsize=None)
def _sc_kernels():
    mesh = plsc.VectorSubcoreMesh(core_axis_name="c", subcore_axis_name="s")

    @functools.partial(
        pl.kernel,
        out_type=jax.ShapeDtypeStruct((PADN, DIM // 2), jnp.int32),
        mesh=mesh,
        scratch_types=[
            pltpu.VMEM((TPW,), jnp.int32),
            pltpu.VMEM((TPW,), jnp.int32),
            pltpu.VMEM((TPW, DIM // 2), jnp.int32),
            pltpu.SemaphoreType.DMA,
        ],
    )
    def _dispatch(x_hbm, pos0_hbm, pos1_hbm, out_hbm, d0_v, d1_v, rows_v, sem):
        wid = lax.axis_index("s") * NC + lax.axis_index("c")
        base = wid * TPW
        pltpu.sync_copy(pos0_hbm.at[pl.ds(base, TPW)], d0_v)
        pltpu.sync_copy(pos1_hbm.at[pl.ds(base, TPW)], d1_v)
        pltpu.sync_copy(x_hbm.at[pl.ds(base, TPW)], rows_v)
        c0 = pltpu.async_copy(rows_v, out_hbm.at[d0_v], sem)
        c1 = pltpu.async_copy(rows_v, out_hbm.at[d1_v], sem)
        c0.wait()
        c1.wait()

    @functools.partial(
        pl.kernel,
        out_type=jax.ShapeDtypeStruct((SEQ, DIM), jnp.float32),
        mesh=mesh,
        scratch_types=[
            pltpu.VMEM((CT,), jnp.int32),
            pltpu.VMEM((CT,), jnp.int32),
            pltpu.VMEM((CT, LW), jnp.float32),
            pltpu.VMEM((CT, LW), jnp.float32),
            pltpu.VMEM((CT, DIM // 2), jnp.int32),
            pltpu.VMEM((CT, DIM // 2), jnp.int32),
            pltpu.VMEM((CT, DIM), jnp.float32),
            pltpu.SemaphoreType.DMA,
            pltpu.SemaphoreType.DMA,
        ],
        compiler_params=pltpu.CompilerParams(needs_layout_passes=False),
    )
    def _comb(ffn_hbm, pos0_hbm, pos1_hbm, w0_hbm, w1_hbm, out_hbm,
              i0_v, i1_v, w0_v, w1_v, ra_v, rb_v, ro_v, sema, semb):
        wid = lax.axis_index("s") * NC + lax.axis_index("c")
        for c in range(TPW // CT):
            base = wid * TPW + c * CT
            pltpu.sync_copy(pos0_hbm.at[pl.ds(base, CT)], i0_v)
            pltpu.sync_copy(pos1_hbm.at[pl.ds(base, CT)], i1_v)
            pltpu.sync_copy(w0_hbm.at[pl.ds(base, CT)], w0_v)
            pltpu.sync_copy(w1_hbm.at[pl.ds(base, CT)], w1_v)
            ca = pltpu.async_copy(ffn_hbm.at[i0_v], ra_v, sema)
            cb = pltpu.async_copy(ffn_hbm.at[i1_v], rb_v, semb)
            ca.wait()
            cb.wait()

            @plsc.parallel_loop(0, CT)
            def _tok(j):
                wa = w0_v[j]
                wb = w1_v[j]
                for k in range(DIM // 2 // LW):
                    sl = pl.ds(k * LW, LW)
                    alo, ahi = plsc.unpack(
                        plsc.bitcast(ra_v[j, sl], jnp.bfloat16),
                        format=plsc.PackFormat.INTERLEAVED)
                    blo, bhi = plsc.unpack(
                        plsc.bitcast(rb_v[j, sl], jnp.bfloat16),
                        format=plsc.PackFormat.INTERLEAVED)
                    ro_v[j, sl] = wa * alo + wb * blo
                    ro_v[j, pl.ds(DIM // 2 + k * LW, LW)] = wa * ahi + wb * bhi

            pltpu.sync_copy(ro_v, out_hbm.at[pl.ds(base, CT)])

    return _dispatch, _comb


def _gemm_body(eid_ref, nact_ref, x_ref, wg_ref, wu_ref, wd_ref, o_ref):
    t = pl.program_id(0)

    @pl.when(t < nact_ref[0])
    def _():
        xw = x_ref[...]
        lo = lax.bitcast_convert_type(
            xw.astype(jnp.int16), jnp.bfloat16).astype(jnp.float32)
        hi = lax.bitcast_convert_type(
            (xw >> 16).astype(jnp.int16), jnp.bfloat16).astype(jnp.float32)
        xt = jnp.concatenate([lo, hi], axis=1)
        g = lax.dot_general(xt, wg_ref[0], (((1,), (1,)), ((), ())),
                            preferred_element_type=jnp.float32)
        u = lax.dot_general(xt, wu_ref[0], (((1,), (1,)), ((), ())),
                            preferred_element_type=jnp.float32)
        h = g * (1.0 / (1.0 + jnp.exp(-g))) * u
        o = lax.dot_general(h, wd_ref[0], (((1,), (1,)), ((), ())),
                            preferred_element_type=jnp.float32)
        olo = lax.bitcast_convert_type(
            o[:, :DIM // 2].astype(jnp.bfloat16), jnp.int16).astype(jnp.int32)
        ohi = lax.bitcast_convert_type(
            o[:, DIM // 2:].astype(jnp.bfloat16), jnp.int16).astype(jnp.int32)
        o_ref[...] = (olo & 0xFFFF) | (ohi << 16)


_gemm = pl.pallas_call(
    _gemm_body,
    grid_spec=pltpu.PrefetchScalarGridSpec(
        num_scalar_prefetch=2,
        grid=(NT,),
        in_specs=[
            pl.BlockSpec((T, DIM // 2),
                         lambda t, eid, na: (jnp.minimum(t, na[0] - 1), 0)),
            pl.BlockSpec((1, FFN, DIM), lambda t, eid, na: (eid[t], 0, 0)),
            pl.BlockSpec((1, FFN, DIM), lambda t, eid, na: (eid[t], 0, 0)),
            pl.BlockSpec((1, DIM, FFN), lambda t, eid, na: (eid[t], 0, 0)),
        ],
        out_specs=pl.BlockSpec(
            (T, DIM // 2),
            lambda t, eid, na: (jnp.where(t < na[0], t, NT - 1), 0)),
    ),
    out_shape=jax.ShapeDtypeStruct((PADN, DIM // 2), jnp.int32),
    compiler_params=pltpu.CompilerParams(
        dimension_semantics=("arbitrary",)),
)


def kernel(hidden_states, gate_weight, gate_proj_w, up_proj_w, down_proj_w):
    b, s, h = hidden_states.shape
    x = hidden_states.reshape(SEQ, DIM).astype(jnp.float32)
    xp, pos0, pos1, wexp, eid2, nact2 = _route(x, gate_weight)
    pos0 = pos0.reshape(SEQ)
    pos1 = pos1.reshape(SEQ)
    _dispatch, _comb = _sc_kernels()
    sorted_x = _dispatch(xp, pos0, pos1)
    ffn = _gemm(eid2.reshape(NT), nact2.reshape(1), sorted_x,
                gate_proj_w, up_proj_w, down_proj_w)
    out = _comb(ffn, pos0, pos1, wexp[:, :LW], wexp[:, LW:])
    return out.reshape(b, s, h)
